# baseline (speedup 1.0000x reference)
import functools

import jax
import jax.numpy as jnp
from jax import lax
from jax.experimental import pallas as pl
from jax.experimental.pallas import tpu as pltpu
from jax.experimental.pallas import tpu_sc as plsc

_STATE = 128
_N_IO = 320000
_N_VAL = 160000

_IO_BLK = 8000
_IO_STEPS = _N_IO // _IO_BLK


def _io_sum_kernel(io_ref, out_ref, acc):
    i = pl.program_id(0)

    @pl.when(i == 0)
    def _init():
        acc[...] = jnp.zeros_like(acc)

    acc[...] += jnp.sum(io_ref[...], axis=0, keepdims=True)

    @pl.when(i == _IO_STEPS - 1)
    def _finish():
        out_ref[...] = acc[...]


def _io_sum(io_embed):
    return pl.pallas_call(
        _io_sum_kernel,
        grid=(_IO_STEPS,),
        in_specs=[pl.BlockSpec((_IO_BLK, 2 * _STATE), lambda i: (i, 0))],
        out_specs=pl.BlockSpec((1, 2 * _STATE), lambda i: (0, 0)),
        out_shape=jax.ShapeDtypeStruct((1, 2 * _STATE), jnp.float32),
        scratch_shapes=[pltpu.VMEM((1, 2 * _STATE), jnp.float32)],
    )(io_embed)


_NC = 2
_NS = 16
_NW = _NC * _NS
_VAL_PER_W = _N_VAL // _NW
_VAL_CHUNK = 200
_VAL_NCHUNK = _VAL_PER_W // _VAL_CHUNK
_ROW_UNROLL = 10


def _sc_val_kernel(val_hbm, out_hbm, buf0, buf1, accv, sem0, sem1):
    c = lax.axis_index("c")
    s = lax.axis_index("s")
    wid = s * _NC + c
    base = wid * _VAL_PER_W
    bufs = (buf0, buf1)
    sems = (sem0, sem1)

    copies = []
    for k in range(2):
        copies.append(
            pltpu.async_copy(
                val_hbm.at[pl.ds(base + k * _VAL_CHUNK, _VAL_CHUNK)],
                bufs[k], sems[k]))

    acc = tuple(jnp.zeros((16,), jnp.float32) for _ in range(8))

    for i in range(_VAL_NCHUNK):
        buf = bufs[i % 2]
        copies[i % 2].wait()

        def body(r, acc):
            accs = list(acc)
            for u in range(_ROW_UNROLL):
                row = r * _ROW_UNROLL + u
                for j in range(8):
                    accs[j] = accs[j] + buf[row, pl.ds(16 * j, 16)]
            return tuple(accs)

        acc = lax.fori_loop(0, _VAL_CHUNK // _ROW_UNROLL, body, acc)

        if i + 2 < _VAL_NCHUNK:
            copies[i % 2] = pltpu.async_copy(
                val_hbm.at[pl.ds(base + (i + 2) * _VAL_CHUNK, _VAL_CHUNK)],
                bufs[i % 2], sems[i % 2])

    for j in range(8):
        accv[pl.ds(16 * j, 16)] = acc[j]
    pltpu.sync_copy(accv, out_hbm.at[wid])


def _val_partials(value_embed):
    mesh = plsc.VectorSubcoreMesh(core_axis_name="c", subcore_axis_name="s")
    run = functools.partial(
        pl.kernel,
        mesh=mesh,
        out_type=jax.ShapeDtypeStruct((_NW, _STATE), jnp.float32),
        scratch_types=[
            pltpu.VMEM((_VAL_CHUNK, _STATE), jnp.float32),
            pltpu.VMEM((_VAL_CHUNK, _STATE), jnp.float32),
            pltpu.VMEM((_STATE,), jnp.float32),
            pltpu.SemaphoreType.DMA,
            pltpu.SemaphoreType.DMA,
        ],
    )(_sc_val_kernel)
    return run(value_embed)


def _combine_kernel(io_sum_ref, val_part_ref, w_ref, b_ref, out_ref):
    io_mean = io_sum_ref[...] / _N_IO
    val_mean = jnp.sum(val_part_ref[...], axis=0, keepdims=True) / _N_VAL
    joint = jnp.concatenate([io_mean, val_mean], axis=1)
    out_ref[...] = (
        lax.dot_general(joint, w_ref[...], (((1,), (1,)), ((), ())),
                        preferred_element_type=jnp.float32)
        + b_ref[...]
    )


def _combine(io_sum, val_partials, W, b2):
    return pl.pallas_call(
        _combine_kernel,
        out_shape=jax.ShapeDtypeStruct((1, _STATE), jnp.float32),
    )(io_sum, val_partials, W, b2)


def kernel(io_embed, value_embed, W, b):
    val_parts = _val_partials(value_embed)
    io_sum = _io_sum(io_embed)
    return _combine(io_sum, val_parts, W, b.reshape(1, _STATE))
